# Initial kernel scaffold; baseline (speedup 1.0000x reference)
import jax
import jax.numpy as jnp
from jax.experimental import pallas as pl


def kernel(x, W_qkv, b_qkv, W_out, b_out):
    raise NotImplementedError("write your pallas kernel here")


# speedup vs baseline: 22.2833x; 22.2833x over previous
import functools
import math

import jax
import jax.numpy as jnp
import numpy as np
from jax.experimental import pallas as pl
from jax.experimental.pallas import tpu as pltpu

DIM = 768
HEADS = 12
HEAD_DIM = 64
K_NEIGH = 64
SCALE = 1.0 / math.sqrt(HEAD_DIM)
NEG = -1e30


@functools.lru_cache(maxsize=None)
def _route_mask_np(seq_len: int, k: int, depth: int = 8):
    pos = np.arange(seq_len)
    x = pos.astype(np.float32) / np.float32(max(1, seq_len - 1))
    x = np.clip(x, np.float32(1e-06), np.float32(1.0 - 1e-06)).astype(np.float32)
    val = np.zeros_like(x)
    factor = 0.5
    for _ in range(depth):
        x_scaled = x * np.float32(3.0)
        digit = x_scaled.astype(np.int32)
        x_frac = (x_scaled - digit.astype(np.float32)).astype(np.float32)
        val = (val + (digit == 2).astype(np.float32) * np.float32(factor)).astype(np.float32)
        x = x_frac
        factor *= 0.5
    val = np.clip(val, 0.0, 1.0).astype(np.float32)
    dist = np.abs(val[:, None] - val[None, :])
    routes = np.argsort(dist, axis=1, kind="stable")[:, :k]
    bias = np.full((seq_len, seq_len), NEG, dtype=np.float32)
    bias[np.arange(seq_len)[:, None], routes] = 0.0
    return bias


def _head_kernel(x_ref, wq_ref, wk_ref, wv_ref, bq_ref, bk_ref, bv_ref,
                 bias_ref, wo_ref, bo_ref, o_ref):
    h = pl.program_id(0)
    x = x_ref[...]
    q = jnp.dot(x, wq_ref[0].T, preferred_element_type=jnp.float32) + bq_ref[0]
    k = jnp.dot(x, wk_ref[0].T, preferred_element_type=jnp.float32) + bk_ref[0]
    v = jnp.dot(x, wv_ref[0].T, preferred_element_type=jnp.float32) + bv_ref[0]
    s = jnp.dot(q, k.T, preferred_element_type=jnp.float32) * SCALE + bias_ref[...]
    m = jnp.max(s, axis=-1, keepdims=True)
    e = jnp.exp(s - m)
    p = e / jnp.sum(e, axis=-1, keepdims=True)
    o = jnp.dot(p, v, preferred_element_type=jnp.float32)
    contrib = jnp.dot(o, wo_ref[0], preferred_element_type=jnp.float32)

    @pl.when(h == 0)
    def _init():
        o_ref[...] = contrib + bo_ref[...]

    @pl.when(h != 0)
    def _acc():
        o_ref[...] = o_ref[...] + contrib


def kernel(x, W_qkv, b_qkv, W_out, b_out):
    B, S, D = x.shape
    H, hd = HEADS, HEAD_DIM
    bias = jnp.asarray(_route_mask_np(S, K_NEIGH))

    x2 = x.reshape(B * S, D)
    w = W_qkv.reshape(3, H, hd, D)
    b = b_qkv.reshape(3, H, 1, hd)
    wo_t = W_out.T.reshape(H, hd, D)
    bo = b_out.reshape(1, D)

    full = lambda *dims: pl.BlockSpec(dims, lambda h: (0,) * len(dims))
    perh = lambda *dims: pl.BlockSpec((1,) + dims, lambda h: (h,) + (0,) * len(dims))

    out = pl.pallas_call(
        _head_kernel,
        grid=(H,),
        in_specs=[
            full(B * S, D),
            perh(hd, D),
            perh(hd, D),
            perh(hd, D),
            perh(1, hd),
            perh(1, hd),
            perh(1, hd),
            full(S, S),
            perh(hd, D),
            full(1, D),
        ],
        out_specs=full(B * S, D),
        out_shape=jax.ShapeDtypeStruct((B * S, D), jnp.float32),
    )(x2, w[0], w[1], w[2], b[0], b[1], b[2], bias, wo_t, bo)
    return out.reshape(B, S, D)
